# Initial kernel scaffold; baseline (speedup 1.0000x reference)
import jax
import jax.numpy as jnp
from jax.experimental import pallas as pl


def kernel(x, edge_index, query_vector, W1, b1, Wg1, bg1, Wg2, bg2, Wpre, bpre, Wpost, bpost, Wlin, blin):
    raise NotImplementedError("write your pallas kernel here")


# speedup vs baseline: 1.1932x; 1.1932x over previous
import math
import functools

import jax
import jax.numpy as jnp
from jax.experimental import pallas as pl

N_NODES = 10000
F = 128
AVG_DEG_LOG = math.log(33.0)
BN = 1000


def _pre_body(x_ref, q_ref, W1_ref, b1_ref, Wg1_ref, bg1_ref, Wg2_ref,
              bg2_ref, Wpre_ref, bpre_ref,
              xg_ref, gs_ref, a_ref, b_ref, b2_ref):
    x = x_ref[...]
    q = q_ref[...]
    W1a = W1_ref[:F, :]
    W1b = W1_ref[F:, :]
    qw = jnp.dot(q, W1b, preferred_element_type=jnp.float32)
    gi = jnp.dot(x, W1a, preferred_element_type=jnp.float32) + qw + b1_ref[...]
    gi = jax.nn.relu(gi)
    h = jax.nn.relu(jnp.dot(gi, Wg1_ref[...],
                            preferred_element_type=jnp.float32) + bg1_ref[...])
    logits = jnp.dot(h, Wg2_ref[...], preferred_element_type=jnp.float32)
    gs = jax.nn.sigmoid(logits + bg2_ref[...])
    xg = x * gs
    a = jnp.dot(xg, Wpre_ref[:F, :],
                preferred_element_type=jnp.float32) + bpre_ref[...]
    b = jnp.dot(xg, Wpre_ref[F:, :], preferred_element_type=jnp.float32)
    xg_ref[...] = xg
    gs_ref[...] = gs
    a_ref[...] = a
    b_ref[...] = b
    b2_ref[...] = b * b


def _pre_stage(x, q, W1, b1, Wg1, bg1, Wg2, bg2, Wpre, bpre):
    n = x.shape[0]
    grid = n // BN
    full = lambda shape: pl.BlockSpec(shape, lambda i: (0, 0))
    outs = [
        jax.ShapeDtypeStruct((n, F), jnp.float32),
        jax.ShapeDtypeStruct((n, 1), jnp.float32),
        jax.ShapeDtypeStruct((n, F), jnp.float32),
        jax.ShapeDtypeStruct((n, F), jnp.float32),
        jax.ShapeDtypeStruct((n, F), jnp.float32),
    ]
    blk = lambda w: pl.BlockSpec((BN, w), lambda i: (i, 0))
    return pl.pallas_call(
        _pre_body,
        grid=(grid,),
        in_specs=[
            blk(F),
            full((1, F)),
            full((2 * F, F)),
            full((1, F)),
            full((F, F)),
            full((1, F)),
            full((F, 1)),
            full((1, 1)),
            full((2 * F, F)),
            full((1, F)),
        ],
        out_specs=[blk(F), blk(1), blk(F), blk(F), blk(F)],
        out_shape=outs,
    )(x, q, W1, b1.reshape(1, F), Wg1, bg1.reshape(1, F), Wg2,
      bg2.reshape(1, 1), Wpre, bpre.reshape(1, F))


def _post_body(xg_ref, a_ref, sb_ref, sb2_ref, mn_ref, mx_ref, deg_ref,
               x_ref, Wpost_ref, bpost_ref, Wlin_ref, blin_ref, out_ref):
    deg = deg_ref[...]
    has = deg > 0.0
    deg_c = jnp.maximum(deg, 1.0)
    inv = 1.0 / deg_c
    a = a_ref[...]
    sb = sb_ref[...]
    eb = sb * inv
    mean = jnp.where(has, a + eb, 0.0)
    mn = jnp.where(has, a + mn_ref[...], 0.0)
    mx = jnp.where(has, a + mx_ref[...], 0.0)
    var = sb2_ref[...] * inv - eb * eb
    std = jnp.sqrt(jax.nn.relu(var) + 1e-5)
    ldeg = jnp.log(deg_c + 1.0)
    amp = ldeg * (1.0 / AVG_DEG_LOG)
    att = AVG_DEG_LOG / ldeg

    def mm(v, r0):
        return jnp.dot(v, Wpost_ref[r0:r0 + F, :],
                       preferred_element_type=jnp.float32)

    comps = (mean, mn, mx, std)
    h = jnp.dot(xg_ref[...], Wpost_ref[:F, :],
                preferred_element_type=jnp.float32)
    acc_id = h
    for k, v in enumerate(comps):
        acc_id += mm(v, F + k * F)
    acc_amp = mm(comps[0], 5 * F)
    for k in range(1, 4):
        acc_amp += mm(comps[k], (5 + k) * F)
    acc_att = mm(comps[0], 9 * F)
    for k in range(1, 4):
        acc_att += mm(comps[k], (9 + k) * F)
    h = acc_id + amp * acc_amp + att * acc_att + bpost_ref[...]
    out = jnp.dot(h, Wlin_ref[...], preferred_element_type=jnp.float32)
    out_ref[...] = out + blin_ref[...] + x_ref[...]


def _post_stage(xg, a, sb, sb2, mn, mx, deg, x, Wpost, bpost, Wlin, blin):
    n = x.shape[0]
    grid = n // BN
    full = lambda shape: pl.BlockSpec(shape, lambda i: (0, 0))
    blk = lambda w: pl.BlockSpec((BN, w), lambda i: (i, 0))
    return pl.pallas_call(
        _post_body,
        grid=(grid,),
        in_specs=[
            blk(F), blk(F), blk(F), blk(F), blk(F), blk(F), blk(1), blk(F),
            full((13 * F, F)),
            full((1, F)),
            full((F, F)),
            full((1, F)),
        ],
        out_specs=blk(F),
        out_shape=jax.ShapeDtypeStruct((n, F), jnp.float32),
    )(xg, a, sb, sb2, mn, mx, deg, x, Wpost, bpost.reshape(1, F), Wlin,
      blin.reshape(1, F))


def _edge_stage(b, b2, src, dst):
    n = b.shape[0]
    bs = b[src]
    sb = jax.ops.segment_sum(bs, dst, num_segments=n)
    sb2 = jax.ops.segment_sum(b2[src], dst, num_segments=n)
    mn = jax.ops.segment_min(bs, dst, num_segments=n)
    mx = jax.ops.segment_max(bs, dst, num_segments=n)
    deg = jax.ops.segment_sum(jnp.ones((src.shape[0],), jnp.float32), dst,
                              num_segments=n)
    return sb, sb2, mn, mx, deg


def kernel(x, edge_index, query_vector, W1, b1, Wg1, bg1, Wg2, bg2,
           Wpre, bpre, Wpost, bpost, Wlin, blin):
    xg, gs, a, b, b2 = _pre_stage(x, query_vector, W1, b1, Wg1, bg1, Wg2,
                                  bg2, Wpre, bpre)
    src = edge_index[0]
    dst = edge_index[1]
    sb, sb2, mn, mx, deg = _edge_stage(b, b2, src, dst)
    out = _post_stage(xg, a, sb, sb2, mn, mx, deg.reshape(-1, 1), x,
                      Wpost, bpost, Wlin, blin)
    return out, gs


# speedup vs baseline: 2.1119x; 1.7699x over previous
import math
import functools

import jax
import jax.numpy as jnp
from jax import lax
from jax.experimental import pallas as pl
from jax.experimental.pallas import tpu as pltpu
from jax.experimental.pallas import tpu_sc as plsc

N_NODES = 10000
F = 128
AVG_DEG_LOG = math.log(33.0)
BN = 1000

N_EDGE = 320000
NW = 32
R = 160
NPASS = 2
NPAD = NW * R * NPASS
GTH = 128
IDXC = 6400
NCHUNK = N_EDGE // IDXC
NGRP = IDXC // 16
TRASH = GTH + 16


def _edge_body(b_hbm, src_hbm, dst_hbm,
               sb_hbm, sb2_hbm, mn_hbm, mx_hbm, deg_hbm,
               accsb, accsb2, accmn, accmx, degacc,
               srcc, dstc, sstage, dstage, gidx, rowsb, sem):
    cid = lax.axis_index("c")
    sid = lax.axis_index("s")
    iota16 = lax.iota(jnp.int32, 16)
    inf16 = jnp.full((16,), jnp.inf, jnp.float32)
    zero16 = jnp.zeros((16,), jnp.float32)
    e0f = (iota16 == 0).astype(jnp.float32)
    lane15 = jnp.full((16,), 15, jnp.int32)

    def process(count):
        for g in range(8):
            pos = iota16 + g * 16
            sv = sstage[pl.ds(g * 16, 16)]
            gidx[pl.ds(g * 16, 16)] = jnp.where(pos < count, sv, 0)
        pltpu.async_copy(b_hbm.at[gidx], rowsb, sem).wait()

        def ent(i, _):
            dv = dstage[pl.ds(i, 16)]
            dl = dv[0]
            dw = degacc[pl.ds(dl, 16)]
            degacc[pl.ds(dl, 16)] = dw + e0f
            for j in range(8):
                sl = pl.ds(j * 16, 16)
                row = rowsb[i, sl]
                accmn[dl, sl] = jnp.minimum(accmn[dl, sl], row)
                accmx[dl, sl] = jnp.maximum(accmx[dl, sl], row)
                accsb[dl, sl] = accsb[dl, sl] + row
                accsb2[dl, sl] = accsb2[dl, sl] + row * row
            return 0
        lax.fori_loop(0, count, ent, 0)

    for p in range(NPASS):
        lo = (p * NW + cid * 16 + sid) * R

        def initacc(r, _):
            for j in range(8):
                sl = pl.ds(j * 16, 16)
                accmn[r, sl] = inf16
                accmx[r, sl] = -inf16
                accsb[r, sl] = zero16
                accsb2[r, sl] = zero16
            return 0
        lax.fori_loop(0, R, initacc, 0)

        def initdeg(r, _):
            degacc[pl.ds(r * 16, 16)] = zero16
            return 0
        lax.fori_loop(0, (R + 16) // 16, initdeg, 0)

        def chunk(ci, kkv):
            pltpu.sync_copy(src_hbm.at[pl.ds(ci * IDXC, IDXC)], srcc)
            pltpu.sync_copy(dst_hbm.at[pl.ds(ci * IDXC, IDXC)], dstc)

            def grp(g, kkv):
                d16 = dstc[pl.ds(g * 16, 16)]
                s16 = srcc[pl.ds(g * 16, 16)]
                m = (d16 >= lo) & (d16 < lo + R)
                mi = m.astype(jnp.int32)
                s = mi
                for k in (1, 2, 4, 8):
                    sh = s.at[jnp.maximum(iota16 - k, 0)].get(
                        mode="promise_in_bounds")
                    s = s + jnp.where(iota16 >= k, sh, 0)
                tot = s.at[lane15].get(mode="promise_in_bounds")
                posv = kkv + s - mi
                wpos = jnp.where(m, posv, TRASH)
                plsc.store_scatter(dstage, [wpos], d16 - lo)
                plsc.store_scatter(sstage, [wpos], s16)
                kkv = kkv + tot
                full = kkv[0] >= GTH

                @pl.when(full)
                def _():
                    process(GTH)
                    dstage[pl.ds(0, 16)] = dstage[pl.ds(GTH, 16)]
                    sstage[pl.ds(0, 16)] = sstage[pl.ds(GTH, 16)]
                return jnp.where(full, kkv - GTH, kkv)
            return lax.fori_loop(0, NGRP, grp, kkv)

        kkv = lax.fori_loop(0, NCHUNK, chunk, jnp.zeros((16,), jnp.int32))
        kk = kkv[0]

        @pl.when(kk > 0)
        def _():
            process(kk)

        pltpu.sync_copy(accsb, sb_hbm.at[pl.ds(lo, R)])
        pltpu.sync_copy(accsb2, sb2_hbm.at[pl.ds(lo, R)])
        pltpu.sync_copy(accmn, mn_hbm.at[pl.ds(lo, R)])
        pltpu.sync_copy(accmx, mx_hbm.at[pl.ds(lo, R)])
        pltpu.sync_copy(degacc.at[pl.ds(0, R)], deg_hbm.at[pl.ds(lo, R)])


def _edge_stage(b, src, dst):
    mesh = plsc.VectorSubcoreMesh(core_axis_name="c", subcore_axis_name="s")
    f = pl.kernel(
        _edge_body,
        mesh=mesh,
        compiler_params=pltpu.CompilerParams(needs_layout_passes=False),
        out_type=[
            jax.ShapeDtypeStruct((NPAD, F), jnp.float32),
            jax.ShapeDtypeStruct((NPAD, F), jnp.float32),
            jax.ShapeDtypeStruct((NPAD, F), jnp.float32),
            jax.ShapeDtypeStruct((NPAD, F), jnp.float32),
            jax.ShapeDtypeStruct((NPAD,), jnp.float32),
        ],
        scratch_types=[
            pltpu.VMEM((R, F), jnp.float32),
            pltpu.VMEM((R, F), jnp.float32),
            pltpu.VMEM((R, F), jnp.float32),
            pltpu.VMEM((R, F), jnp.float32),
            pltpu.VMEM((R + 16,), jnp.float32),
            pltpu.VMEM((IDXC,), jnp.int32),
            pltpu.VMEM((IDXC,), jnp.int32),
            pltpu.VMEM((GTH + 32,), jnp.int32),
            pltpu.VMEM((GTH + 32,), jnp.int32),
            pltpu.VMEM((GTH,), jnp.int32),
            pltpu.VMEM((GTH, F), jnp.float32),
            pltpu.SemaphoreType.DMA,
        ],
    )
    sb, sb2, mn, mx, deg = f(b, src, dst)
    n = b.shape[0]
    return sb[:n], sb2[:n], mn[:n], mx[:n], deg[:n]


def _pre_body(x_ref, q_ref, W1_ref, b1_ref, Wg1_ref, bg1_ref, Wg2_ref,
              bg2_ref, Wpre_ref, bpre_ref,
              xg_ref, gs_ref, a_ref, b_ref):
    x = x_ref[...]
    q = q_ref[...]
    W1a = W1_ref[:F, :]
    W1b = W1_ref[F:, :]
    qw = jnp.dot(q, W1b, preferred_element_type=jnp.float32)
    gi = jnp.dot(x, W1a, preferred_element_type=jnp.float32) + qw + b1_ref[...]
    gi = jax.nn.relu(gi)
    h = jax.nn.relu(jnp.dot(gi, Wg1_ref[...],
                            preferred_element_type=jnp.float32) + bg1_ref[...])
    logits = jnp.dot(h, Wg2_ref[...], preferred_element_type=jnp.float32)
    gs = jax.nn.sigmoid(logits + bg2_ref[...])
    xg = x * gs
    a = jnp.dot(xg, Wpre_ref[:F, :],
                preferred_element_type=jnp.float32) + bpre_ref[...]
    b = jnp.dot(xg, Wpre_ref[F:, :], preferred_element_type=jnp.float32)
    xg_ref[...] = xg
    gs_ref[...] = gs
    a_ref[...] = a
    b_ref[...] = b


def _pre_stage(x, q, W1, b1, Wg1, bg1, Wg2, bg2, Wpre, bpre):
    n = x.shape[0]
    grid = n // BN
    full = lambda shape: pl.BlockSpec(shape, lambda i: (0, 0))
    outs = [
        jax.ShapeDtypeStruct((n, F), jnp.float32),
        jax.ShapeDtypeStruct((n, 1), jnp.float32),
        jax.ShapeDtypeStruct((n, F), jnp.float32),
        jax.ShapeDtypeStruct((n, F), jnp.float32),
    ]
    blk = lambda w: pl.BlockSpec((BN, w), lambda i: (i, 0))
    return pl.pallas_call(
        _pre_body,
        grid=(grid,),
        in_specs=[
            blk(F),
            full((1, F)),
            full((2 * F, F)),
            full((1, F)),
            full((F, F)),
            full((1, F)),
            full((F, 1)),
            full((1, 1)),
            full((2 * F, F)),
            full((1, F)),
        ],
        out_specs=[blk(F), blk(1), blk(F), blk(F)],
        out_shape=outs,
    )(x, q, W1, b1.reshape(1, F), Wg1, bg1.reshape(1, F), Wg2,
      bg2.reshape(1, 1), Wpre, bpre.reshape(1, F))


def _post_body(xg_ref, a_ref, sb_ref, sb2_ref, mn_ref, mx_ref, deg_ref,
               x_ref, Wpost_ref, bpost_ref, Wlin_ref, blin_ref, out_ref):
    deg = deg_ref[...]
    has = deg > 0.0
    deg_c = jnp.maximum(deg, 1.0)
    inv = 1.0 / deg_c
    a = a_ref[...]
    sb = sb_ref[...]
    eb = sb * inv
    mean = jnp.where(has, a + eb, 0.0)
    mn = jnp.where(has, a + mn_ref[...], 0.0)
    mx = jnp.where(has, a + mx_ref[...], 0.0)
    var = sb2_ref[...] * inv - eb * eb
    std = jnp.sqrt(jax.nn.relu(var) + 1e-5)
    ldeg = jnp.log(deg_c + 1.0)
    amp = ldeg * (1.0 / AVG_DEG_LOG)
    att = AVG_DEG_LOG / ldeg

    def mm(v, r0):
        return jnp.dot(v, Wpost_ref[r0:r0 + F, :],
                       preferred_element_type=jnp.float32)

    comps = (mean, mn, mx, std)
    h = jnp.dot(xg_ref[...], Wpost_ref[:F, :],
                preferred_element_type=jnp.float32)
    acc_id = h
    for k, v in enumerate(comps):
        acc_id += mm(v, F + k * F)
    acc_amp = mm(comps[0], 5 * F)
    for k in range(1, 4):
        acc_amp += mm(comps[k], (5 + k) * F)
    acc_att = mm(comps[0], 9 * F)
    for k in range(1, 4):
        acc_att += mm(comps[k], (9 + k) * F)
    h = acc_id + amp * acc_amp + att * acc_att + bpost_ref[...]
    out = jnp.dot(h, Wlin_ref[...], preferred_element_type=jnp.float32)
    out_ref[...] = out + blin_ref[...] + x_ref[...]


def _post_stage(xg, a, sb, sb2, mn, mx, deg, x, Wpost, bpost, Wlin, blin):
    n = x.shape[0]
    grid = n // BN
    full = lambda shape: pl.BlockSpec(shape, lambda i: (0, 0))
    blk = lambda w: pl.BlockSpec((BN, w), lambda i: (i, 0))
    return pl.pallas_call(
        _post_body,
        grid=(grid,),
        in_specs=[
            blk(F), blk(F), blk(F), blk(F), blk(F), blk(F), blk(1), blk(F),
            full((13 * F, F)),
            full((1, F)),
            full((F, F)),
            full((1, F)),
        ],
        out_specs=blk(F),
        out_shape=jax.ShapeDtypeStruct((n, F), jnp.float32),
    )(xg, a, sb, sb2, mn, mx, deg, x, Wpost, bpost.reshape(1, F), Wlin,
      blin.reshape(1, F))


def kernel(x, edge_index, query_vector, W1, b1, Wg1, bg1, Wg2, bg2,
           Wpre, bpre, Wpost, bpost, Wlin, blin):
    xg, gs, a, b = _pre_stage(x, query_vector, W1, b1, Wg1, bg1, Wg2,
                              bg2, Wpre, bpre)
    src = edge_index[0]
    dst = edge_index[1]
    sb, sb2, mn, mx, deg = _edge_stage(b, src, dst)
    out = _post_stage(xg, a, sb, sb2, mn, mx, deg.reshape(-1, 1), x,
                      Wpost, bpost, Wlin, blin)
    return out, gs


# speedup vs baseline: 2.8516x; 1.3503x over previous
import math
import functools

import jax
import jax.numpy as jnp
from jax import lax
from jax.experimental import pallas as pl
from jax.experimental.pallas import tpu as pltpu
from jax.experimental.pallas import tpu_sc as plsc

N_NODES = 10000
F = 128
AVG_DEG_LOG = math.log(33.0)
BN = 1000

N_EDGE = 320000
NW = 32
R = 160
NPASS = 2
NPAD = NW * R * NPASS
GTH = 128
IDXC = 6400
NCHUNK = N_EDGE // IDXC
NGRP2 = IDXC // 32
TRASH = GTH + 32
STAGE = GTH + 48


def _edge_body(b_hbm, src_hbm, dst_hbm,
               sb_hbm, sb2_hbm, mn_hbm, mx_hbm, deg_hbm,
               accsb, accsb2, accmn, accmx, degacc,
               srcc, dstc, sstage, dstage, gidx, rowsb, sem):
    cid = lax.axis_index("c")
    sid = lax.axis_index("s")
    iota16 = lax.iota(jnp.int32, 16)
    inf16 = jnp.full((16,), jnp.inf, jnp.float32)
    zero16 = jnp.zeros((16,), jnp.float32)
    e0f = (iota16 == 0).astype(jnp.float32)
    lane15 = jnp.full((16,), 15, jnp.int32)

    def process(count):
        for g in range(8):
            pos = iota16 + g * 16
            sv = sstage[pl.ds(g * 16, 16)]
            gidx[pl.ds(g * 16, 16)] = jnp.where(pos < count, sv, 0)
        pltpu.async_copy(b_hbm.at[gidx], rowsb, sem).wait()

        def ent(i, _):
            dv = dstage[pl.ds(i, 16)]
            dl = dv[0]
            dw = degacc[pl.ds(dl, 16)]
            degacc[pl.ds(dl, 16)] = dw + e0f
            for j in range(8):
                sl = pl.ds(j * 16, 16)
                row = rowsb[i, sl]
                accmn[dl, sl] = jnp.minimum(accmn[dl, sl], row)
                accmx[dl, sl] = jnp.maximum(accmx[dl, sl], row)
                accsb[dl, sl] = accsb[dl, sl] + row
                accsb2[dl, sl] = accsb2[dl, sl] + row * row
            return 0
        lax.fori_loop(0, count, ent, 0)

    for p in range(NPASS):
        lo = (p * NW + cid * 16 + sid) * R

        def initacc(r, _):
            for j in range(8):
                sl = pl.ds(j * 16, 16)
                accmn[r, sl] = inf16
                accmx[r, sl] = -inf16
                accsb[r, sl] = zero16
                accsb2[r, sl] = zero16
            return 0
        lax.fori_loop(0, R, initacc, 0)

        def initdeg(r, _):
            degacc[pl.ds(r * 16, 16)] = zero16
            return 0
        lax.fori_loop(0, (R + 16) // 16, initdeg, 0)

        def chunk(ci, kkv):
            pltpu.sync_copy(src_hbm.at[pl.ds(ci * IDXC, IDXC)], srcc)
            pltpu.sync_copy(dst_hbm.at[pl.ds(ci * IDXC, IDXC)], dstc)

            def prefix(m):
                s = m.astype(jnp.int32)
                for k in (1, 2, 4, 8):
                    sh = s.at[jnp.maximum(iota16 - k, 0)].get(
                        mode="promise_in_bounds")
                    s = s + jnp.where(iota16 >= k, sh, 0)
                return s

            def grp(g, kkv):
                base = g * 32
                dA = dstc[pl.ds(base, 16)]
                dB = dstc[pl.ds(base + 16, 16)]
                sA = srcc[pl.ds(base, 16)]
                sB = srcc[pl.ds(base + 16, 16)]
                mA = (dA >= lo) & (dA < lo + R)
                mB = (dB >= lo) & (dB < lo + R)
                pA = prefix(mA)
                pB = prefix(mB)
                totA = pA.at[lane15].get(mode="promise_in_bounds")
                totB = pB.at[lane15].get(mode="promise_in_bounds")
                posA = kkv + pA - mA.astype(jnp.int32)
                posB = kkv + totA + pB - mB.astype(jnp.int32)
                plsc.store_scatter(dstage, [jnp.where(mA, posA, TRASH)],
                                   dA - lo)
                plsc.store_scatter(sstage, [jnp.where(mA, posA, TRASH)], sA)
                plsc.store_scatter(dstage, [jnp.where(mB, posB, TRASH)],
                                   dB - lo)
                plsc.store_scatter(sstage, [jnp.where(mB, posB, TRASH)], sB)
                kkv = kkv + totA + totB
                full = kkv[0] >= GTH

                @pl.when(full)
                def _():
                    process(GTH)
                    dstage[pl.ds(0, 16)] = dstage[pl.ds(GTH, 16)]
                    sstage[pl.ds(0, 16)] = sstage[pl.ds(GTH, 16)]
                    dstage[pl.ds(16, 16)] = dstage[pl.ds(GTH + 16, 16)]
                    sstage[pl.ds(16, 16)] = sstage[pl.ds(GTH + 16, 16)]
                return jnp.where(full, kkv - GTH, kkv)
            return lax.fori_loop(0, NGRP2, grp, kkv)

        kkv = lax.fori_loop(0, NCHUNK, chunk, jnp.zeros((16,), jnp.int32))
        kk = kkv[0]

        @pl.when(kk > 0)
        def _():
            process(kk)

        pltpu.sync_copy(accsb, sb_hbm.at[pl.ds(lo, R)])
        pltpu.sync_copy(accsb2, sb2_hbm.at[pl.ds(lo, R)])
        pltpu.sync_copy(accmn, mn_hbm.at[pl.ds(lo, R)])
        pltpu.sync_copy(accmx, mx_hbm.at[pl.ds(lo, R)])
        pltpu.sync_copy(degacc.at[pl.ds(0, R)], deg_hbm.at[pl.ds(lo, R)])


def _edge_stage(b, src, dst):
    mesh = plsc.VectorSubcoreMesh(core_axis_name="c", subcore_axis_name="s")
    f = pl.kernel(
        _edge_body,
        mesh=mesh,
        compiler_params=pltpu.CompilerParams(needs_layout_passes=False),
        out_type=[
            jax.ShapeDtypeStruct((NPAD, F), jnp.float32),
            jax.ShapeDtypeStruct((NPAD, F), jnp.float32),
            jax.ShapeDtypeStruct((NPAD, F), jnp.float32),
            jax.ShapeDtypeStruct((NPAD, F), jnp.float32),
            jax.ShapeDtypeStruct((NPAD,), jnp.float32),
        ],
        scratch_types=[
            pltpu.VMEM((R, F), jnp.float32),
            pltpu.VMEM((R, F), jnp.float32),
            pltpu.VMEM((R, F), jnp.float32),
            pltpu.VMEM((R, F), jnp.float32),
            pltpu.VMEM((R + 16,), jnp.float32),
            pltpu.VMEM((IDXC,), jnp.int32),
            pltpu.VMEM((IDXC,), jnp.int32),
            pltpu.VMEM((STAGE,), jnp.int32),
            pltpu.VMEM((STAGE,), jnp.int32),
            pltpu.VMEM((GTH,), jnp.int32),
            pltpu.VMEM((GTH, F), jnp.float32),
            pltpu.SemaphoreType.DMA,
        ],
    )
    sb, sb2, mn, mx, deg = f(b, src, dst)
    n = b.shape[0]
    return sb[:n], sb2[:n], mn[:n], mx[:n], deg[:n]


def _pre_body(x_ref, q_ref, W1_ref, b1_ref, Wg1_ref, bg1_ref, Wg2_ref,
              bg2_ref, Wpre_ref, bpre_ref,
              xg_ref, gs_ref, a_ref, b_ref):
    x = x_ref[...]
    q = q_ref[...]
    W1a = W1_ref[:F, :]
    W1b = W1_ref[F:, :]
    qw = jnp.dot(q, W1b, preferred_element_type=jnp.float32)
    gi = jnp.dot(x, W1a, preferred_element_type=jnp.float32) + qw + b1_ref[...]
    gi = jax.nn.relu(gi)
    h = jax.nn.relu(jnp.dot(gi, Wg1_ref[...],
                            preferred_element_type=jnp.float32) + bg1_ref[...])
    logits = jnp.dot(h, Wg2_ref[...], preferred_element_type=jnp.float32)
    gs = jax.nn.sigmoid(logits + bg2_ref[...])
    xg = x * gs
    a = jnp.dot(xg, Wpre_ref[:F, :],
                preferred_element_type=jnp.float32) + bpre_ref[...]
    b = jnp.dot(xg, Wpre_ref[F:, :], preferred_element_type=jnp.float32)
    xg_ref[...] = xg
    gs_ref[...] = gs
    a_ref[...] = a
    b_ref[...] = b


def _pre_stage(x, q, W1, b1, Wg1, bg1, Wg2, bg2, Wpre, bpre):
    n = x.shape[0]
    grid = n // BN
    full = lambda shape: pl.BlockSpec(shape, lambda i: (0, 0))
    outs = [
        jax.ShapeDtypeStruct((n, F), jnp.float32),
        jax.ShapeDtypeStruct((n, 1), jnp.float32),
        jax.ShapeDtypeStruct((n, F), jnp.float32),
        jax.ShapeDtypeStruct((n, F), jnp.float32),
    ]
    blk = lambda w: pl.BlockSpec((BN, w), lambda i: (i, 0))
    return pl.pallas_call(
        _pre_body,
        grid=(grid,),
        in_specs=[
            blk(F),
            full((1, F)),
            full((2 * F, F)),
            full((1, F)),
            full((F, F)),
            full((1, F)),
            full((F, 1)),
            full((1, 1)),
            full((2 * F, F)),
            full((1, F)),
        ],
        out_specs=[blk(F), blk(1), blk(F), blk(F)],
        out_shape=outs,
    )(x, q, W1, b1.reshape(1, F), Wg1, bg1.reshape(1, F), Wg2,
      bg2.reshape(1, 1), Wpre, bpre.reshape(1, F))


def _post_body(xg_ref, a_ref, sb_ref, sb2_ref, mn_ref, mx_ref, deg_ref,
               x_ref, Wpost_ref, bpost_ref, Wlin_ref, blin_ref, out_ref):
    deg = deg_ref[...]
    has = deg > 0.0
    deg_c = jnp.maximum(deg, 1.0)
    inv = 1.0 / deg_c
    a = a_ref[...]
    sb = sb_ref[...]
    eb = sb * inv
    mean = jnp.where(has, a + eb, 0.0)
    mn = jnp.where(has, a + mn_ref[...], 0.0)
    mx = jnp.where(has, a + mx_ref[...], 0.0)
    var = sb2_ref[...] * inv - eb * eb
    std = jnp.sqrt(jax.nn.relu(var) + 1e-5)
    ldeg = jnp.log(deg_c + 1.0)
    amp = ldeg * (1.0 / AVG_DEG_LOG)
    att = AVG_DEG_LOG / ldeg

    def mm(v, r0):
        return jnp.dot(v, Wpost_ref[r0:r0 + F, :],
                       preferred_element_type=jnp.float32)

    comps = (mean, mn, mx, std)
    h = jnp.dot(xg_ref[...], Wpost_ref[:F, :],
                preferred_element_type=jnp.float32)
    acc_id = h
    for k, v in enumerate(comps):
        acc_id += mm(v, F + k * F)
    acc_amp = mm(comps[0], 5 * F)
    for k in range(1, 4):
        acc_amp += mm(comps[k], (5 + k) * F)
    acc_att = mm(comps[0], 9 * F)
    for k in range(1, 4):
        acc_att += mm(comps[k], (9 + k) * F)
    h = acc_id + amp * acc_amp + att * acc_att + bpost_ref[...]
    out = jnp.dot(h, Wlin_ref[...], preferred_element_type=jnp.float32)
    out_ref[...] = out + blin_ref[...] + x_ref[...]


def _post_stage(xg, a, sb, sb2, mn, mx, deg, x, Wpost, bpost, Wlin, blin):
    n = x.shape[0]
    grid = n // BN
    full = lambda shape: pl.BlockSpec(shape, lambda i: (0, 0))
    blk = lambda w: pl.BlockSpec((BN, w), lambda i: (i, 0))
    return pl.pallas_call(
        _post_body,
        grid=(grid,),
        in_specs=[
            blk(F), blk(F), blk(F), blk(F), blk(F), blk(F), blk(1), blk(F),
            full((13 * F, F)),
            full((1, F)),
            full((F, F)),
            full((1, F)),
        ],
        out_specs=blk(F),
        out_shape=jax.ShapeDtypeStruct((n, F), jnp.float32),
    )(xg, a, sb, sb2, mn, mx, deg, x, Wpost, bpost.reshape(1, F), Wlin,
      blin.reshape(1, F))


def kernel(x, edge_index, query_vector, W1, b1, Wg1, bg1, Wg2, bg2,
           Wpre, bpre, Wpost, bpost, Wlin, blin):
    xg, gs, a, b = _pre_stage(x, query_vector, W1, b1, Wg1, bg1, Wg2,
                              bg2, Wpre, bpre)
    src = edge_index[0]
    dst = edge_index[1]
    sb, sb2, mn, mx, deg = _edge_stage(b, src, dst)
    out = _post_stage(xg, a, sb, sb2, mn, mx, deg.reshape(-1, 1), x,
                      Wpost, bpost, Wlin, blin)
    return out, gs


# speedup vs baseline: 3.8300x; 1.3431x over previous
import math
import functools

import jax
import jax.numpy as jnp
from jax import lax
from jax.experimental import pallas as pl
from jax.experimental.pallas import tpu as pltpu
from jax.experimental.pallas import tpu_sc as plsc

N_NODES = 10000
F = 128
AVG_DEG_LOG = math.log(33.0)
BN = 1000

N_EDGE = 320000
NW = 32
R = 160
NPASS = 2
NPAD = NW * R * NPASS
GTH = 128
IDXC = 6400
NCHUNK = N_EDGE // IDXC
NGRP4 = IDXC // 64
TRASH = GTH + 64
STAGE = GTH + 80


def _edge_body(b_hbm, src_hbm, dst_hbm,
               sb_hbm, sb2_hbm, mn_hbm, mx_hbm, deg_hbm,
               accsb, accsb2, accmn, accmx, degacc,
               srcc, dstc, srcc1, dstc1, sstage, dstage, gidx, rowsb,
               sem, semS0, semD0, semS1, semD1):
    cid = lax.axis_index("c")
    sid = lax.axis_index("s")
    iota16 = lax.iota(jnp.int32, 16)
    inf16 = jnp.full((16,), jnp.inf, jnp.float32)
    zero16 = jnp.zeros((16,), jnp.float32)
    e0f = (iota16 == 0).astype(jnp.float32)
    lane15 = jnp.full((16,), 15, jnp.int32)

    def process(count):
        for g in range(8):
            pos = iota16 + g * 16
            sv = sstage[pl.ds(g * 16, 16)]
            gidx[pl.ds(g * 16, 16)] = jnp.where(pos < count, sv, 0)
        pltpu.async_copy(b_hbm.at[gidx], rowsb, sem).wait()

        def ent(i, _):
            dv = dstage[pl.ds(i, 16)]
            dl = dv[0]
            dw = degacc[pl.ds(dl, 16)]
            degacc[pl.ds(dl, 16)] = dw + e0f
            for j in range(8):
                sl = pl.ds(j * 16, 16)
                row = rowsb[i, sl]
                accmn[dl, sl] = jnp.minimum(accmn[dl, sl], row)
                accmx[dl, sl] = jnp.maximum(accmx[dl, sl], row)
                accsb[dl, sl] = accsb[dl, sl] + row
                accsb2[dl, sl] = accsb2[dl, sl] + row * row
            return 0
        lax.fori_loop(0, count, ent, 0)

    for p in range(NPASS):
        lo = (p * NW + cid * 16 + sid) * R

        def initacc(r, _):
            for j in range(8):
                sl = pl.ds(j * 16, 16)
                accmn[r, sl] = inf16
                accmx[r, sl] = -inf16
                accsb[r, sl] = zero16
                accsb2[r, sl] = zero16
            return 0
        lax.fori_loop(0, R, initacc, 0)

        def initdeg(r, _):
            degacc[pl.ds(r * 16, 16)] = zero16
            return 0
        lax.fori_loop(0, (R + 16) // 16, initdeg, 0)

        def prefix(m):
            s = m.astype(jnp.int32)
            for k in (1, 2, 4, 8):
                sh = s.at[jnp.maximum(iota16 - k, 0)].get(
                    mode="promise_in_bounds")
                s = s + jnp.where(iota16 >= k, sh, 0)
            return s

        def scan_buf(sc, dc, kkv):
            def grp(g, kkv):
                base = g * 64
                run = kkv
                wd, ws = [], []
                for q in range(4):
                    dq = dc[pl.ds(base + q * 16, 16)]
                    sq = sc[pl.ds(base + q * 16, 16)]
                    mq = (dq >= lo) & (dq < lo + R)
                    pq = prefix(mq)
                    posq = run + pq - mq.astype(jnp.int32)
                    run = run + pq.at[lane15].get(mode="promise_in_bounds")
                    wd.append((jnp.where(mq, posq, TRASH), dq - lo))
                    ws.append((jnp.where(mq, posq, TRASH), sq))
                for wp, v in wd:
                    plsc.store_scatter(dstage, [wp], v)
                for wp, v in ws:
                    plsc.store_scatter(sstage, [wp], v)
                kkv = run
                full = kkv[0] >= GTH

                @pl.when(full)
                def _():
                    process(GTH)
                    for t in range(4):
                        dstage[pl.ds(t * 16, 16)] = (
                            dstage[pl.ds(GTH + t * 16, 16)])
                        sstage[pl.ds(t * 16, 16)] = (
                            sstage[pl.ds(GTH + t * 16, 16)])
                return jnp.where(full, kkv - GTH, kkv)
            return lax.fori_loop(0, NGRP4, grp, kkv)

        pltpu.async_copy(src_hbm.at[pl.ds(0, IDXC)], srcc, semS0)
        pltpu.async_copy(dst_hbm.at[pl.ds(0, IDXC)], dstc, semD0)

        def chunk2(cc, kkv):
            ci = cc * 2
            pltpu.make_async_copy(
                src_hbm.at[pl.ds(ci * IDXC, IDXC)], srcc, semS0).wait()
            pltpu.make_async_copy(
                dst_hbm.at[pl.ds(ci * IDXC, IDXC)], dstc, semD0).wait()
            pltpu.async_copy(
                src_hbm.at[pl.ds((ci + 1) * IDXC, IDXC)], srcc1, semS1)
            pltpu.async_copy(
                dst_hbm.at[pl.ds((ci + 1) * IDXC, IDXC)], dstc1, semD1)
            kkv = scan_buf(srcc, dstc, kkv)
            pltpu.make_async_copy(
                src_hbm.at[pl.ds((ci + 1) * IDXC, IDXC)], srcc1, semS1).wait()
            pltpu.make_async_copy(
                dst_hbm.at[pl.ds((ci + 1) * IDXC, IDXC)], dstc1, semD1).wait()

            @pl.when(cc + 1 < NCHUNK // 2)
            def _():
                pltpu.async_copy(
                    src_hbm.at[pl.ds((ci + 2) * IDXC, IDXC)], srcc, semS0)
                pltpu.async_copy(
                    dst_hbm.at[pl.ds((ci + 2) * IDXC, IDXC)], dstc, semD0)
            return scan_buf(srcc1, dstc1, kkv)

        kkv = lax.fori_loop(0, NCHUNK // 2, chunk2,
                            jnp.zeros((16,), jnp.int32))
        kk = kkv[0]

        @pl.when(kk > 0)
        def _():
            process(kk)

        pltpu.sync_copy(accsb, sb_hbm.at[pl.ds(lo, R)])
        pltpu.sync_copy(accsb2, sb2_hbm.at[pl.ds(lo, R)])
        pltpu.sync_copy(accmn, mn_hbm.at[pl.ds(lo, R)])
        pltpu.sync_copy(accmx, mx_hbm.at[pl.ds(lo, R)])
        pltpu.sync_copy(degacc.at[pl.ds(0, R)], deg_hbm.at[pl.ds(lo, R)])


def _edge_stage(b, src, dst):
    mesh = plsc.VectorSubcoreMesh(core_axis_name="c", subcore_axis_name="s")
    f = pl.kernel(
        _edge_body,
        mesh=mesh,
        compiler_params=pltpu.CompilerParams(needs_layout_passes=False),
        out_type=[
            jax.ShapeDtypeStruct((NPAD, F), jnp.float32),
            jax.ShapeDtypeStruct((NPAD, F), jnp.float32),
            jax.ShapeDtypeStruct((NPAD, F), jnp.float32),
            jax.ShapeDtypeStruct((NPAD, F), jnp.float32),
            jax.ShapeDtypeStruct((NPAD,), jnp.float32),
        ],
        scratch_types=[
            pltpu.VMEM((R, F), jnp.float32),
            pltpu.VMEM((R, F), jnp.float32),
            pltpu.VMEM((R, F), jnp.float32),
            pltpu.VMEM((R, F), jnp.float32),
            pltpu.VMEM((R + 16,), jnp.float32),
            pltpu.VMEM((IDXC,), jnp.int32),
            pltpu.VMEM((IDXC,), jnp.int32),
            pltpu.VMEM((IDXC,), jnp.int32),
            pltpu.VMEM((IDXC,), jnp.int32),
            pltpu.VMEM((STAGE,), jnp.int32),
            pltpu.VMEM((STAGE,), jnp.int32),
            pltpu.VMEM((GTH,), jnp.int32),
            pltpu.VMEM((GTH, F), jnp.float32),
            pltpu.SemaphoreType.DMA,
            pltpu.SemaphoreType.DMA,
            pltpu.SemaphoreType.DMA,
            pltpu.SemaphoreType.DMA,
            pltpu.SemaphoreType.DMA,
        ],
    )
    sb, sb2, mn, mx, deg = f(b, src, dst)
    n = b.shape[0]
    return sb[:n], sb2[:n], mn[:n], mx[:n], deg[:n]


def _pre_body(x_ref, q_ref, W1_ref, b1_ref, Wg1_ref, bg1_ref, Wg2_ref,
              bg2_ref, Wpre_ref, bpre_ref,
              xg_ref, gs_ref, a_ref, b_ref):
    x = x_ref[...]
    q = q_ref[...]
    W1a = W1_ref[:F, :]
    W1b = W1_ref[F:, :]
    qw = jnp.dot(q, W1b, preferred_element_type=jnp.float32)
    gi = jnp.dot(x, W1a, preferred_element_type=jnp.float32) + qw + b1_ref[...]
    gi = jax.nn.relu(gi)
    h = jax.nn.relu(jnp.dot(gi, Wg1_ref[...],
                            preferred_element_type=jnp.float32) + bg1_ref[...])
    logits = jnp.dot(h, Wg2_ref[...], preferred_element_type=jnp.float32)
    gs = jax.nn.sigmoid(logits + bg2_ref[...])
    xg = x * gs
    a = jnp.dot(xg, Wpre_ref[:F, :],
                preferred_element_type=jnp.float32) + bpre_ref[...]
    b = jnp.dot(xg, Wpre_ref[F:, :], preferred_element_type=jnp.float32)
    xg_ref[...] = xg
    gs_ref[...] = gs
    a_ref[...] = a
    b_ref[...] = b


def _pre_stage(x, q, W1, b1, Wg1, bg1, Wg2, bg2, Wpre, bpre):
    n = x.shape[0]
    grid = n // BN
    full = lambda shape: pl.BlockSpec(shape, lambda i: (0, 0))
    outs = [
        jax.ShapeDtypeStruct((n, F), jnp.float32),
        jax.ShapeDtypeStruct((n, 1), jnp.float32),
        jax.ShapeDtypeStruct((n, F), jnp.float32),
        jax.ShapeDtypeStruct((n, F), jnp.float32),
    ]
    blk = lambda w: pl.BlockSpec((BN, w), lambda i: (i, 0))
    return pl.pallas_call(
        _pre_body,
        grid=(grid,),
        in_specs=[
            blk(F),
            full((1, F)),
            full((2 * F, F)),
            full((1, F)),
            full((F, F)),
            full((1, F)),
            full((F, 1)),
            full((1, 1)),
            full((2 * F, F)),
            full((1, F)),
        ],
        out_specs=[blk(F), blk(1), blk(F), blk(F)],
        out_shape=outs,
    )(x, q, W1, b1.reshape(1, F), Wg1, bg1.reshape(1, F), Wg2,
      bg2.reshape(1, 1), Wpre, bpre.reshape(1, F))


def _post_body(xg_ref, a_ref, sb_ref, sb2_ref, mn_ref, mx_ref, deg_ref,
               x_ref, Wpost_ref, bpost_ref, Wlin_ref, blin_ref, out_ref):
    deg = deg_ref[...]
    has = deg > 0.0
    deg_c = jnp.maximum(deg, 1.0)
    inv = 1.0 / deg_c
    a = a_ref[...]
    sb = sb_ref[...]
    eb = sb * inv
    mean = jnp.where(has, a + eb, 0.0)
    mn = jnp.where(has, a + mn_ref[...], 0.0)
    mx = jnp.where(has, a + mx_ref[...], 0.0)
    var = sb2_ref[...] * inv - eb * eb
    std = jnp.sqrt(jax.nn.relu(var) + 1e-5)
    ldeg = jnp.log(deg_c + 1.0)
    amp = ldeg * (1.0 / AVG_DEG_LOG)
    att = AVG_DEG_LOG / ldeg

    def mm(v, r0):
        return jnp.dot(v, Wpost_ref[r0:r0 + F, :],
                       preferred_element_type=jnp.float32)

    comps = (mean, mn, mx, std)
    h = jnp.dot(xg_ref[...], Wpost_ref[:F, :],
                preferred_element_type=jnp.float32)
    acc_id = h
    for k, v in enumerate(comps):
        acc_id += mm(v, F + k * F)
    acc_amp = mm(comps[0], 5 * F)
    for k in range(1, 4):
        acc_amp += mm(comps[k], (5 + k) * F)
    acc_att = mm(comps[0], 9 * F)
    for k in range(1, 4):
        acc_att += mm(comps[k], (9 + k) * F)
    h = acc_id + amp * acc_amp + att * acc_att + bpost_ref[...]
    out = jnp.dot(h, Wlin_ref[...], preferred_element_type=jnp.float32)
    out_ref[...] = out + blin_ref[...] + x_ref[...]


def _post_stage(xg, a, sb, sb2, mn, mx, deg, x, Wpost, bpost, Wlin, blin):
    n = x.shape[0]
    grid = n // BN
    full = lambda shape: pl.BlockSpec(shape, lambda i: (0, 0))
    blk = lambda w: pl.BlockSpec((BN, w), lambda i: (i, 0))
    return pl.pallas_call(
        _post_body,
        grid=(grid,),
        in_specs=[
            blk(F), blk(F), blk(F), blk(F), blk(F), blk(F), blk(1), blk(F),
            full((13 * F, F)),
            full((1, F)),
            full((F, F)),
            full((1, F)),
        ],
        out_specs=blk(F),
        out_shape=jax.ShapeDtypeStruct((n, F), jnp.float32),
    )(xg, a, sb, sb2, mn, mx, deg, x, Wpost, bpost.reshape(1, F), Wlin,
      blin.reshape(1, F))


def kernel(x, edge_index, query_vector, W1, b1, Wg1, bg1, Wg2, bg2,
           Wpre, bpre, Wpost, bpost, Wlin, blin):
    xg, gs, a, b = _pre_stage(x, query_vector, W1, b1, Wg1, bg1, Wg2,
                              bg2, Wpre, bpre)
    src = edge_index[0]
    dst = edge_index[1]
    sb, sb2, mn, mx, deg = _edge_stage(b, src, dst)
    out = _post_stage(xg, a, sb, sb2, mn, mx, deg.reshape(-1, 1), x,
                      Wpost, bpost, Wlin, blin)
    return out, gs
